# baseline (speedup 1.0000x reference)
import functools

import jax
import jax.numpy as jnp
import numpy as np
from jax import lax
from jax.experimental import pallas as pl
from jax.experimental.pallas import tpu as pltpu
from jax.experimental.pallas import tpu_sc as plsc

N_LEVELS = 6
N_LABELS = 2048
BATCH = 1024
ROW = 2048
ROW_W = ROW // 8

NC = 2
NS = 16
NW = NC * NS
B_PER_W = BATCH // NW
ROWS_PER_W = B_PER_W * N_LEVELS
CHUNK = 16
NCHUNK = ROWS_PER_W // CHUNK
NBUF = 3

_PACK = np.zeros((ROW, ROW_W), np.int8)
for _c in range(ROW):
    _PACK[_c, _c % ROW_W] = np.int8(np.uint8(1 << (_c // ROW_W)))
_SHIFTS = np.repeat(np.arange(8, dtype=np.int32), ROW_W)


def _body(leaf_hbm, adv_hbm, out_hbm, leaf_v, gidx_v,
          rows0, rows1, rows2, gsem0, gsem1, gsem2):
    wid = lax.axis_index("s") * NC + lax.axis_index("c")
    b0 = wid * B_PER_W
    pltpu.sync_copy(leaf_hbm.at[pl.ds(b0, B_PER_W)], leaf_v)

    leaf_half = (leaf_v[pl.ds(0, 16)], leaf_v[pl.ds(16, 16)])
    rows_bufs = (rows0, rows1, rows2)
    gsems = (gsem0, gsem1, gsem2)

    for c in range(NCHUNK):
        gidx_v[c, pl.ds(0, 16)] = leaf_half[c % 2] + (c // 2) * N_LABELS

    copies = [None] * NBUF
    for c in range(NCHUNK + NBUF):
        s = c % NBUF
        if c >= NBUF:
            d = c - NBUF
            copies[s].wait()
            out_base = (d // 2) * BATCH + b0 + (d % 2) * CHUNK
            pltpu.sync_copy(rows_bufs[s], out_hbm.at[pl.ds(out_base, CHUNK)])
        if c < NCHUNK:
            copies[s] = pltpu.async_copy(
                adv_hbm.at[gidx_v.at[c]], rows_bufs[s], gsems[s]
            )


@jax.jit
def _sc_gather(leaf, adv_packed):
    mesh = plsc.VectorSubcoreMesh(core_axis_name="c", subcore_axis_name="s")
    f = functools.partial(
        pl.kernel,
        mesh=mesh,
        out_type=jax.ShapeDtypeStruct((N_LEVELS * BATCH, ROW_W), jnp.int32),
        scratch_types=[
            pltpu.VMEM((B_PER_W,), jnp.int32),
            pltpu.VMEM((NCHUNK, CHUNK), jnp.int32),
            pltpu.VMEM((CHUNK, ROW_W), jnp.int32),
            pltpu.VMEM((CHUNK, ROW_W), jnp.int32),
            pltpu.VMEM((CHUNK, ROW_W), jnp.int32),
            pltpu.SemaphoreType.DMA,
            pltpu.SemaphoreType.DMA,
            pltpu.SemaphoreType.DMA,
        ],
    )(_body)
    return f(leaf, adv_packed)


def kernel(labels, adversaries):
    leaf = labels[:, N_LEVELS - 1]
    a8 = adversaries.astype(jnp.int8).reshape(N_LEVELS * N_LABELS, ROW)
    packed = lax.dot_general(
        a8, jnp.asarray(_PACK), (((1,), (0,)), ((), ())),
        preferred_element_type=jnp.int32,
    )
    g = _sc_gather(leaf, packed)
    y = (jnp.tile(g, (1, 8)) >> jnp.asarray(_SHIFTS)) & 1
    out = (y != 0).reshape(N_LEVELS, BATCH, ROW)
    return out.transpose(1, 0, 2)


# speedup vs baseline: 2.0075x; 2.0075x over previous
import functools

import jax
import jax.numpy as jnp
import numpy as np
from jax import lax
from jax.experimental import pallas as pl
from jax.experimental.pallas import tpu as pltpu
from jax.experimental.pallas import tpu_sc as plsc

N_LEVELS = 6
N_LABELS = 2048
BATCH = 1024
ROW = 2048
ROW_W = ROW // 8

NC = 2
NS = 16
NW = NC * NS
B_PER_W = BATCH // NW
ROWS_PER_W = B_PER_W * N_LEVELS
CHUNK = 16
NCHUNK = ROWS_PER_W // CHUNK
NBUF = 3

_PACK = np.zeros((ROW, ROW_W), np.int8)
for _c in range(ROW):
    _PACK[_c, _c % ROW_W] = np.int8(np.uint8(1 << (_c // ROW_W)))
_SHIFTS = np.repeat(np.arange(8, dtype=np.int32), ROW_W)


def _body(leaf_hbm, adv_hbm, out_hbm, leaf_v, gidx_v,
          rows0, rows1, rows2, gsem0, gsem1, gsem2):
    wid = lax.axis_index("s") * NC + lax.axis_index("c")
    b0 = wid * B_PER_W
    pltpu.sync_copy(leaf_hbm.at[pl.ds(b0, B_PER_W)], leaf_v)

    leaf_half = (leaf_v[pl.ds(0, 16)], leaf_v[pl.ds(16, 16)])
    rows_bufs = (rows0, rows1, rows2)
    gsems = (gsem0, gsem1, gsem2)

    for c in range(NCHUNK):
        gidx_v[c, pl.ds(0, 16)] = leaf_half[c % 2] + (c // 2) * N_LABELS

    copies = [None] * NBUF
    for c in range(NCHUNK + NBUF):
        s = c % NBUF
        if c >= NBUF:
            d = c - NBUF
            copies[s].wait()
            out_base = (d // 2) * BATCH + b0 + (d % 2) * CHUNK
            pltpu.sync_copy(rows_bufs[s], out_hbm.at[pl.ds(out_base, CHUNK)])
        if c < NCHUNK:
            copies[s] = pltpu.async_copy(
                adv_hbm.at[gidx_v.at[c]], rows_bufs[s], gsems[s]
            )


@jax.jit
def _sc_gather(leaf, adv_packed):
    mesh = plsc.VectorSubcoreMesh(core_axis_name="c", subcore_axis_name="s")
    f = functools.partial(
        pl.kernel,
        mesh=mesh,
        out_type=jax.ShapeDtypeStruct((N_LEVELS * BATCH, ROW_W), jnp.int32),
        scratch_types=[
            pltpu.VMEM((B_PER_W,), jnp.int32),
            pltpu.VMEM((NCHUNK, CHUNK), jnp.int32),
            pltpu.VMEM((CHUNK, ROW_W), jnp.int32),
            pltpu.VMEM((CHUNK, ROW_W), jnp.int32),
            pltpu.VMEM((CHUNK, ROW_W), jnp.int32),
            pltpu.SemaphoreType.DMA,
            pltpu.SemaphoreType.DMA,
            pltpu.SemaphoreType.DMA,
        ],
    )(_body)
    return f(leaf, adv_packed)


UNPACK_BLK = 512


def _unpack_body(g_ref, out_ref):
    x = g_ref[...]
    for k in range(8):
        out_ref[:, pl.ds(k * ROW_W, ROW_W)] = ((x >> k) & 1).astype(jnp.int8)


@jax.jit
def _tc_unpack(g):
    return pl.pallas_call(
        _unpack_body,
        grid=(N_LEVELS * BATCH // UNPACK_BLK,),
        in_specs=[pl.BlockSpec((UNPACK_BLK, ROW_W), lambda i: (i, 0))],
        out_specs=pl.BlockSpec((UNPACK_BLK, ROW), lambda i: (i, 0)),
        out_shape=jax.ShapeDtypeStruct((N_LEVELS * BATCH, ROW), jnp.int8),
    )(g)


def kernel(labels, adversaries):
    leaf = labels[:, N_LEVELS - 1]
    a8 = adversaries.astype(jnp.int8).reshape(N_LEVELS * N_LABELS, ROW)
    packed = lax.dot_general(
        a8, jnp.asarray(_PACK), (((1,), (0,)), ((), ())),
        preferred_element_type=jnp.int32,
    )
    g = _sc_gather(leaf, packed)
    out = (_tc_unpack(g) != 0).reshape(N_LEVELS, BATCH, ROW)
    return out.transpose(1, 0, 2)


# speedup vs baseline: 2.0278x; 1.0101x over previous
import functools

import jax
import jax.numpy as jnp
import numpy as np
from jax import lax
from jax.experimental import pallas as pl
from jax.experimental.pallas import tpu as pltpu
from jax.experimental.pallas import tpu_sc as plsc

N_LEVELS = 6
N_LABELS = 2048
BATCH = 1024
ROW = 2048
ROW_W = ROW // 8

NC = 2
NS = 16
NW = NC * NS
B_PER_W = BATCH // NW
ROWS_PER_W = B_PER_W * N_LEVELS
CHUNK = 16
NCHUNK = ROWS_PER_W // CHUNK
NBUF = 3

_PACK = np.zeros((ROW, ROW_W), np.int8)
for _c in range(ROW):
    _PACK[_c, _c % ROW_W] = np.int8(np.uint8(1 << (_c // ROW_W)))
_SHIFTS = np.repeat(np.arange(8, dtype=np.int32), ROW_W)


def _body(leaf_hbm, adv_hbm, out_hbm, leaf_v, gidx_v,
          rows0, rows1, rows2, gsem0, gsem1, gsem2):
    wid = lax.axis_index("s") * NC + lax.axis_index("c")
    b0 = wid * B_PER_W
    pltpu.sync_copy(leaf_hbm.at[pl.ds(b0, B_PER_W)], leaf_v)

    leaf_half = (leaf_v[pl.ds(0, 16)], leaf_v[pl.ds(16, 16)])
    rows_bufs = (rows0, rows1, rows2)
    gsems = (gsem0, gsem1, gsem2)

    for c in range(NCHUNK):
        gidx_v[c, pl.ds(0, 16)] = leaf_half[c % 2] + (c // 2) * N_LABELS

    copies = [None] * NBUF
    for c in range(NCHUNK + NBUF):
        s = c % NBUF
        if c >= NBUF:
            d = c - NBUF
            copies[s].wait()
            out_base = (d // 2) * BATCH + b0 + (d % 2) * CHUNK
            pltpu.sync_copy(rows_bufs[s], out_hbm.at[pl.ds(out_base, CHUNK)])
        if c < NCHUNK:
            copies[s] = pltpu.async_copy(
                adv_hbm.at[gidx_v.at[c]], rows_bufs[s], gsems[s]
            )


@jax.jit
def _sc_gather(leaf, adv_packed):
    mesh = plsc.VectorSubcoreMesh(core_axis_name="c", subcore_axis_name="s")
    f = functools.partial(
        pl.kernel,
        mesh=mesh,
        out_type=jax.ShapeDtypeStruct((N_LEVELS * BATCH, ROW_W), jnp.int32),
        scratch_types=[
            pltpu.VMEM((B_PER_W,), jnp.int32),
            pltpu.VMEM((NCHUNK, CHUNK), jnp.int32),
            pltpu.VMEM((CHUNK, ROW_W), jnp.int32),
            pltpu.VMEM((CHUNK, ROW_W), jnp.int32),
            pltpu.VMEM((CHUNK, ROW_W), jnp.int32),
            pltpu.SemaphoreType.DMA,
            pltpu.SemaphoreType.DMA,
            pltpu.SemaphoreType.DMA,
        ],
    )(_body)
    return f(leaf, adv_packed)


UNPACK_BLK = 512


def _unpack_body(g_ref, out_ref):
    x = g_ref[...]
    for k in range(8):
        out_ref[:, pl.ds(k * ROW_W, ROW_W)] = ((x >> k) & 1).astype(jnp.int8)


@jax.jit
def _tc_unpack(g):
    return pl.pallas_call(
        _unpack_body,
        grid=(N_LEVELS * BATCH // UNPACK_BLK,),
        in_specs=[pl.BlockSpec((UNPACK_BLK, ROW_W), lambda i: (i, 0))],
        out_specs=pl.BlockSpec((UNPACK_BLK, ROW), lambda i: (i, 0)),
        out_shape=jax.ShapeDtypeStruct((N_LEVELS * BATCH, ROW), jnp.int8),
    )(g)


def kernel(labels, adversaries):
    leaf = labels[:, N_LEVELS - 1]
    a16 = adversaries.astype(jnp.bfloat16).reshape(N_LEVELS * N_LABELS, ROW)
    packed_f = lax.dot_general(
        a16, jnp.asarray(_PACK).astype(jnp.bfloat16), (((1,), (0,)), ((), ())),
        preferred_element_type=jnp.float32,
    )
    packed = packed_f.astype(jnp.int32)
    g = _sc_gather(leaf, packed)
    out = (_tc_unpack(g) != 0).reshape(N_LEVELS, BATCH, ROW)
    return out.transpose(1, 0, 2)


# speedup vs baseline: 2.1119x; 1.0415x over previous
import functools

import jax
import jax.numpy as jnp
import numpy as np
from jax import lax
from jax.experimental import pallas as pl
from jax.experimental.pallas import tpu as pltpu
from jax.experimental.pallas import tpu_sc as plsc

N_LEVELS = 6
N_LABELS = 2048
BATCH = 1024
ROW = 2048
ROW_W = ROW // 8

NC = 2
NS = 16
NW = NC * NS
B_PER_W = BATCH // NW
ROWS_PER_W = B_PER_W * N_LEVELS
CHUNK = 16
NCHUNK = ROWS_PER_W // CHUNK
NBUF = 3

_PACK = np.zeros((ROW, ROW_W), np.int8)
for _c in range(ROW):
    _PACK[_c, _c % ROW_W] = np.int8(np.uint8(1 << (_c // ROW_W)))
_SHIFTS = np.repeat(np.arange(8, dtype=np.int32), ROW_W)


def _body(leaf_hbm, adv_hbm, out_hbm, leaf_v, gidx_v,
          rows0, rows1, rows2, gsem0, gsem1, gsem2):
    wid = lax.axis_index("s") * NC + lax.axis_index("c")
    b0 = wid * B_PER_W
    pltpu.sync_copy(leaf_hbm.at[pl.ds(b0, B_PER_W)], leaf_v)

    leaf_half = (leaf_v[pl.ds(0, 16)], leaf_v[pl.ds(16, 16)])
    rows_bufs = (rows0, rows1, rows2)
    gsems = (gsem0, gsem1, gsem2)

    for c in range(NCHUNK):
        gidx_v[c, pl.ds(0, 16)] = leaf_half[c % 2] + (c // 2) * N_LABELS

    copies = [None] * NBUF
    for c in range(NCHUNK + NBUF):
        s = c % NBUF
        if c >= NBUF:
            d = c - NBUF
            copies[s].wait()
            out_base = (d // 2) * BATCH + b0 + (d % 2) * CHUNK
            pltpu.sync_copy(rows_bufs[s], out_hbm.at[pl.ds(out_base, CHUNK)])
        if c < NCHUNK:
            copies[s] = pltpu.async_copy(
                adv_hbm.at[gidx_v.at[c]], rows_bufs[s], gsems[s]
            )


@jax.jit
def _sc_gather(leaf, adv_packed):
    mesh = plsc.VectorSubcoreMesh(core_axis_name="c", subcore_axis_name="s")
    f = functools.partial(
        pl.kernel,
        mesh=mesh,
        out_type=jax.ShapeDtypeStruct((N_LEVELS * BATCH, ROW_W), jnp.int32),
        scratch_types=[
            pltpu.VMEM((B_PER_W,), jnp.int32),
            pltpu.VMEM((NCHUNK, CHUNK), jnp.int32),
            pltpu.VMEM((CHUNK, ROW_W), jnp.int32),
            pltpu.VMEM((CHUNK, ROW_W), jnp.int32),
            pltpu.VMEM((CHUNK, ROW_W), jnp.int32),
            pltpu.SemaphoreType.DMA,
            pltpu.SemaphoreType.DMA,
            pltpu.SemaphoreType.DMA,
        ],
    )(_body)
    return f(leaf, adv_packed)


UNPACK_BLK = 1024


def _unpack_body(g_ref, out_ref):
    x = g_ref[...]
    for k in range(8):
        out_ref[:, pl.ds(k * ROW_W, ROW_W)] = ((x >> k) & 1).astype(jnp.int8)


@jax.jit
def _tc_unpack(g):
    return pl.pallas_call(
        _unpack_body,
        grid=(N_LEVELS * BATCH // UNPACK_BLK,),
        in_specs=[pl.BlockSpec((UNPACK_BLK, ROW_W), lambda i: (i, 0))],
        out_specs=pl.BlockSpec((UNPACK_BLK, ROW), lambda i: (i, 0)),
        out_shape=jax.ShapeDtypeStruct((N_LEVELS * BATCH, ROW), jnp.int8),
    )(g)


def kernel(labels, adversaries):
    leaf = labels[:, N_LEVELS - 1]
    a16 = adversaries.astype(jnp.bfloat16).reshape(N_LEVELS * N_LABELS, ROW)
    packed_f = lax.dot_general(
        a16, jnp.asarray(_PACK).astype(jnp.bfloat16), (((1,), (0,)), ((), ())),
        preferred_element_type=jnp.float32,
    )
    packed = packed_f.astype(jnp.int32)
    g = _sc_gather(leaf, packed)
    out = (_tc_unpack(g) != 0).reshape(N_LEVELS, BATCH, ROW)
    return out.transpose(1, 0, 2)


# speedup vs baseline: 2.1160x; 1.0019x over previous
import functools

import jax
import jax.numpy as jnp
import numpy as np
from jax import lax
from jax.experimental import pallas as pl
from jax.experimental.pallas import tpu as pltpu
from jax.experimental.pallas import tpu_sc as plsc

N_LEVELS = 6
N_LABELS = 2048
BATCH = 1024
ROW = 2048
ROW_W = ROW // 8

NC = 2
NS = 16
NW = NC * NS
B_PER_W = BATCH // NW
ROWS_PER_W = B_PER_W * N_LEVELS
CHUNK = 16
NCHUNK = ROWS_PER_W // CHUNK
NBUF = 3

_PACK = np.zeros((ROW, ROW_W), np.int8)
for _c in range(ROW):
    _PACK[_c, _c % ROW_W] = np.int8(np.uint8(1 << (_c // ROW_W)))
_SHIFTS = np.repeat(np.arange(8, dtype=np.int32), ROW_W)


def _body(leaf_hbm, adv_hbm, out_hbm, leaf_v, gidx_v,
          rows0, rows1, rows2, gsem0, gsem1, gsem2):
    wid = lax.axis_index("s") * NC + lax.axis_index("c")
    b0 = wid * B_PER_W
    pltpu.sync_copy(leaf_hbm.at[pl.ds(b0, B_PER_W)], leaf_v)

    leaf_half = (leaf_v[pl.ds(0, 16)], leaf_v[pl.ds(16, 16)])
    rows_bufs = (rows0, rows1, rows2)
    gsems = (gsem0, gsem1, gsem2)

    for c in range(NCHUNK):
        gidx_v[c, pl.ds(0, 16)] = leaf_half[c % 2] + (c // 2) * N_LABELS

    copies = [None] * NBUF
    for c in range(NCHUNK + NBUF):
        s = c % NBUF
        if c >= NBUF:
            d = c - NBUF
            copies[s].wait()
            out_base = (d // 2) * BATCH + b0 + (d % 2) * CHUNK
            pltpu.sync_copy(rows_bufs[s], out_hbm.at[pl.ds(out_base, CHUNK)])
        if c < NCHUNK:
            copies[s] = pltpu.async_copy(
                adv_hbm.at[gidx_v.at[c]], rows_bufs[s], gsems[s]
            )


@jax.jit
def _sc_gather(leaf, adv_packed):
    mesh = plsc.VectorSubcoreMesh(core_axis_name="c", subcore_axis_name="s")
    f = functools.partial(
        pl.kernel,
        mesh=mesh,
        out_type=jax.ShapeDtypeStruct((N_LEVELS * BATCH, ROW_W), jnp.int32),
        scratch_types=[
            pltpu.VMEM((B_PER_W,), jnp.int32),
            pltpu.VMEM((NCHUNK, CHUNK), jnp.int32),
            pltpu.VMEM((CHUNK, ROW_W), jnp.int32),
            pltpu.VMEM((CHUNK, ROW_W), jnp.int32),
            pltpu.VMEM((CHUNK, ROW_W), jnp.int32),
            pltpu.SemaphoreType.DMA,
            pltpu.SemaphoreType.DMA,
            pltpu.SemaphoreType.DMA,
        ],
    )(_body)
    return f(leaf, adv_packed)


UNPACK_BLK = 2048


def _unpack_body(g_ref, out_ref):
    x = g_ref[...]
    for k in range(8):
        out_ref[:, pl.ds(k * ROW_W, ROW_W)] = ((x >> k) & 1).astype(jnp.int8)


@jax.jit
def _tc_unpack(g):
    return pl.pallas_call(
        _unpack_body,
        grid=(N_LEVELS * BATCH // UNPACK_BLK,),
        in_specs=[pl.BlockSpec((UNPACK_BLK, ROW_W), lambda i: (i, 0))],
        out_specs=pl.BlockSpec((UNPACK_BLK, ROW), lambda i: (i, 0)),
        out_shape=jax.ShapeDtypeStruct((N_LEVELS * BATCH, ROW), jnp.int8),
    )(g)


def kernel(labels, adversaries):
    leaf = labels[:, N_LEVELS - 1]
    a16 = adversaries.astype(jnp.bfloat16).reshape(N_LEVELS * N_LABELS, ROW)
    packed_f = lax.dot_general(
        a16, jnp.asarray(_PACK).astype(jnp.bfloat16), (((1,), (0,)), ((), ())),
        preferred_element_type=jnp.float32,
    )
    packed = packed_f.astype(jnp.int32)
    g = _sc_gather(leaf, packed)
    out = (_tc_unpack(g) != 0).reshape(N_LEVELS, BATCH, ROW)
    return out.transpose(1, 0, 2)


# speedup vs baseline: 2.1224x; 1.0031x over previous
import functools

import jax
import jax.numpy as jnp
import numpy as np
from jax import lax
from jax.experimental import pallas as pl
from jax.experimental.pallas import tpu as pltpu
from jax.experimental.pallas import tpu_sc as plsc

N_LEVELS = 6
N_LABELS = 2048
BATCH = 1024
ROW = 2048
ROW_W = ROW // 8

NC = 2
NS = 16
NW = NC * NS
B_PER_W = BATCH // NW
ROWS_PER_W = B_PER_W * N_LEVELS
CHUNK = 16
NCHUNK = ROWS_PER_W // CHUNK
NBUF = 3

_PACK = np.zeros((ROW, ROW_W), np.int8)
for _c in range(ROW):
    _PACK[_c, _c % ROW_W] = np.int8(np.uint8(1 << (_c // ROW_W)))


def _body(leaf_hbm, adv_hbm, out_hbm, leaf_v, gidx_v,
          rows0, rows1, rows2, gsem0, gsem1, gsem2):
    wid = lax.axis_index("s") * NC + lax.axis_index("c")
    b0 = wid * B_PER_W
    pltpu.sync_copy(leaf_hbm.at[pl.ds(b0, B_PER_W)], leaf_v)

    leaf_half = (leaf_v[pl.ds(0, 16)], leaf_v[pl.ds(16, 16)])
    rows_bufs = (rows0, rows1, rows2)
    gsems = (gsem0, gsem1, gsem2)

    for c in range(NCHUNK):
        gidx_v[c, pl.ds(0, 16)] = leaf_half[c % 2] + (c // 2) * N_LABELS

    copies = [None] * NBUF
    for c in range(NCHUNK + NBUF):
        s = c % NBUF
        if c >= NBUF:
            d = c - NBUF
            copies[s].wait()
            out_base = (d // 2) * BATCH + b0 + (d % 2) * CHUNK
            pltpu.sync_copy(rows_bufs[s], out_hbm.at[pl.ds(out_base, CHUNK)])
        if c < NCHUNK:
            copies[s] = pltpu.async_copy(
                adv_hbm.at[gidx_v.at[c]], rows_bufs[s], gsems[s]
            )


@jax.jit
def _sc_gather(leaf, adv_packed):
    mesh = plsc.VectorSubcoreMesh(core_axis_name="c", subcore_axis_name="s")
    f = functools.partial(
        pl.kernel,
        mesh=mesh,
        out_type=jax.ShapeDtypeStruct((N_LEVELS * BATCH, ROW_W), jnp.int32),
        scratch_types=[
            pltpu.VMEM((B_PER_W,), jnp.int32),
            pltpu.VMEM((NCHUNK, CHUNK), jnp.int32),
            pltpu.VMEM((CHUNK, ROW_W), jnp.int32),
            pltpu.VMEM((CHUNK, ROW_W), jnp.int32),
            pltpu.VMEM((CHUNK, ROW_W), jnp.int32),
            pltpu.SemaphoreType.DMA,
            pltpu.SemaphoreType.DMA,
            pltpu.SemaphoreType.DMA,
        ],
    )(_body)
    return f(leaf, adv_packed)


UNPACK_BLK = 2048


def _unpack_body(g_ref, out_ref):
    x = g_ref[...]
    for k in range(8):
        out_ref[:, pl.ds(k * ROW_W, ROW_W)] = ((x >> k) & 1).astype(jnp.int8)


@jax.jit
def _tc_unpack(g):
    return pl.pallas_call(
        _unpack_body,
        grid=(N_LEVELS * BATCH // UNPACK_BLK,),
        in_specs=[pl.BlockSpec((UNPACK_BLK, ROW_W), lambda i: (i, 0))],
        out_specs=pl.BlockSpec((UNPACK_BLK, ROW), lambda i: (i, 0)),
        out_shape=jax.ShapeDtypeStruct((N_LEVELS * BATCH, ROW), jnp.int8),
    )(g)


def kernel(labels, adversaries):
    leaf = labels[:, N_LEVELS - 1]
    a16 = adversaries.astype(jnp.bfloat16).reshape(N_LEVELS * N_LABELS, ROW)
    packed_f = lax.dot_general(
        a16, jnp.asarray(_PACK).astype(jnp.bfloat16), (((1,), (0,)), ((), ())),
        preferred_element_type=jnp.float32,
    )
    packed = packed_f.astype(jnp.int32)
    g = _sc_gather(leaf, packed)
    out = (_tc_unpack(g) != 0).reshape(N_LEVELS, BATCH, ROW)
    return out.transpose(1, 0, 2)
